# initial kernel scaffold (unmeasured)
import jax
import jax.numpy as jnp
from jax import lax
from jax.experimental import pallas as pl
from jax.experimental.pallas import tpu as pltpu


def kernel(
    x,
):
    def body(*refs):
        pass

    out_shape = jax.ShapeDtypeStruct(..., jnp.float32)
    return pl.pallas_call(body, out_shape=out_shape)(...)


# baseline (device time: 33497 ns/iter reference)
import jax
import jax.numpy as jnp
from jax import lax
from jax.experimental import pallas as pl
from jax.experimental.pallas import tpu as pltpu

N = 16


def kernel(x):
    m, n = x.shape
    ch = m // N

    def body(x_ref, out_ref, rs_buf, ag_buf, rs_send, rs_recv, ag_send, ag_recv):
        p = lax.axis_index("i")

        rs_rdmas = []
        for d in range(1, N):
            t = (p + d) % N
            rdma = pltpu.make_async_remote_copy(
                src_ref=x_ref.at[pl.ds(t * ch, ch), :],
                dst_ref=rs_buf.at[d],
                send_sem=rs_send.at[d],
                recv_sem=rs_recv.at[d],
                device_id=(t,),
                device_id_type=pl.DeviceIdType.MESH,
            )
            rdma.start()
            rs_rdmas.append(rdma)

        rs_buf[0, :, :] = x_ref[pl.ds(p * ch, ch), :]

        for rdma in rs_rdmas:
            rdma.wait_recv()
        out_ref[pl.ds(p * ch, ch), :] = jnp.sum(rs_buf[...], axis=0)

        ag_rdmas = []
        for d in range(1, N):
            t = (p + d) % N
            rdma = pltpu.make_async_remote_copy(
                src_ref=out_ref.at[pl.ds(p * ch, ch), :],
                dst_ref=ag_buf.at[d],
                send_sem=ag_send.at[d],
                recv_sem=ag_recv.at[d],
                device_id=(t,),
                device_id_type=pl.DeviceIdType.MESH,
            )
            rdma.start()
            ag_rdmas.append(rdma)

        for d in range(1, N):
            ag_rdmas[d - 1].wait_recv()
            out_ref[pl.ds(((p - d) % N) * ch, ch), :] = ag_buf[d, :, :]

        for rdma in rs_rdmas:
            rdma.wait_send()
        for rdma in ag_rdmas:
            rdma.wait_send()

    return pl.pallas_call(
        body,
        out_shape=jax.ShapeDtypeStruct((m, n), x.dtype),
        in_specs=[pl.BlockSpec(memory_space=pltpu.VMEM)],
        out_specs=pl.BlockSpec(memory_space=pltpu.VMEM),
        scratch_shapes=[
            pltpu.VMEM((N, ch, n), x.dtype),
            pltpu.VMEM((N, ch, n), x.dtype),
            pltpu.SemaphoreType.DMA((N,)),
            pltpu.SemaphoreType.DMA((N,)),
            pltpu.SemaphoreType.DMA((N,)),
            pltpu.SemaphoreType.DMA((N,)),
        ],
    )(x)
